# baseline (speedup 1.0000x reference)
import functools

import jax
import jax.numpy as jnp
from jax import lax
from jax.experimental import pallas as pl
from jax.experimental.pallas import tpu as pltpu
from jax.experimental.pallas import tpu_sc as plsc

_N_ROWS = 10000
_N_UNITS = 2
_ROW_BLK = 1000
_LANES = 16


def _lut_body(f_ref, k_ref, b_ref, o_ref):
    rs = jnp.sum(jnp.abs(f_ref[...]), axis=1, keepdims=True)
    o_ref[...] = jax.nn.sigmoid(rs * k_ref[...] + b_ref[...])


def _build_lut(feature2d, kern, bias):
    grid = _N_ROWS // _ROW_BLK
    return pl.pallas_call(
        _lut_body,
        grid=(grid,),
        in_specs=[
            pl.BlockSpec((_ROW_BLK, 128), lambda i: (i, 0)),
            pl.BlockSpec((1, _N_UNITS), lambda i: (0, 0)),
            pl.BlockSpec((1, _N_UNITS), lambda i: (0, 0)),
        ],
        out_specs=pl.BlockSpec((_ROW_BLK, _N_UNITS), lambda i: (i, 0)),
        out_shape=jax.ShapeDtypeStruct((_N_ROWS, _N_UNITS), jnp.float32),
    )(feature2d, kern.reshape(1, _N_UNITS), bias.reshape(1, _N_UNITS))


def _gather_lut(ppi_flat, lut_flat, n_flat):
    info = plsc.get_sparse_core_info()
    nc, ns = info.num_cores, info.num_subcores
    nw = nc * ns
    chunk = n_flat // nw
    lut_n = _N_ROWS * _N_UNITS

    mesh = plsc.VectorSubcoreMesh(core_axis_name="c", subcore_axis_name="s")

    @functools.partial(
        pl.kernel,
        mesh=mesh,
        out_type=jax.ShapeDtypeStruct((n_flat,), jnp.float32),
        scratch_types=[
            pltpu.VMEM((chunk,), jnp.int32),
            pltpu.VMEM((lut_n,), jnp.float32),
            pltpu.VMEM((chunk,), jnp.float32),
        ],
        compiler_params=pltpu.CompilerParams(
            use_tc_tiling_on_sc=False,
            needs_layout_passes=False,
        ),
    )
    def gather_k(ppi_hbm, lut_hbm, out_hbm, idx_v, lut_v, out_v):
        wid = lax.axis_index("s") * nc + lax.axis_index("c")
        base = wid * chunk
        pltpu.sync_copy(lut_hbm, lut_v)
        pltpu.sync_copy(ppi_hbm.at[pl.ds(base, chunk)], idx_v)
        parity = lax.iota(jnp.int32, 16) % _N_UNITS

        def body(i, _):
            idx = idx_v[pl.ds(i * _LANES, _LANES)]
            fidx = idx * _N_UNITS + parity
            out_v[pl.ds(i * _LANES, _LANES)] = plsc.load_gather(lut_v, [fidx])
            return 0

        lax.fori_loop(0, chunk // _LANES, body, 0)
        pltpu.sync_copy(out_v, out_hbm.at[pl.ds(base, chunk)])

    return gather_k(ppi_flat, lut_flat)


def kernel(feature, ppi, kernel, bias):
    n_edges = ppi.shape[0]
    n_flat = n_edges * _N_UNITS
    lut = feature.reshape(-1)[: _N_ROWS * _N_UNITS]
    out_flat = _gather_lut(ppi.reshape(n_flat), lut, n_flat)
    return out_flat.reshape(1, n_edges, _N_UNITS)


# speedup vs baseline: 8.6751x; 8.6751x over previous
import functools

import jax
import jax.numpy as jnp
from jax import lax
from jax.experimental import pallas as pl
from jax.experimental.pallas import tpu as pltpu
from jax.experimental.pallas import tpu_sc as plsc

_N_ROWS = 10000
_N_UNITS = 2
_ROW_BLK = 1000
_LANES = 16


def _lut_body(f_ref, k_ref, b_ref, o_ref):
    rs = jnp.sum(jnp.abs(f_ref[0]), axis=1)
    o_ref[...] = jax.nn.sigmoid(rs[None, :] * k_ref[...] + b_ref[...])


def _build_lut(feature, kern, bias):
    return pl.pallas_call(
        _lut_body,
        out_shape=jax.ShapeDtypeStruct((_N_UNITS, _N_ROWS), jnp.float32),
    )(feature, kern.reshape(_N_UNITS, 1), bias.reshape(_N_UNITS, 1))


def _gather_lut(ppi_t, lut, n_edges):
    info = plsc.get_sparse_core_info()
    nc, ns = info.num_cores, info.num_subcores
    nw = nc * ns
    chunk_e = n_edges // nw

    mesh = plsc.VectorSubcoreMesh(core_axis_name="c", subcore_axis_name="s")

    @functools.partial(
        pl.kernel,
        mesh=mesh,
        out_type=jax.ShapeDtypeStruct((_N_UNITS, n_edges), jnp.float32),
        scratch_types=[
            pltpu.VMEM((chunk_e,), jnp.int32),
            pltpu.VMEM((chunk_e,), jnp.int32),
            pltpu.VMEM((_N_ROWS,), jnp.float32),
            pltpu.VMEM((_N_ROWS,), jnp.float32),
            pltpu.VMEM((chunk_e,), jnp.float32),
            pltpu.VMEM((chunk_e,), jnp.float32),
        ],
        compiler_params=pltpu.CompilerParams(
            use_tc_tiling_on_sc=False,
            needs_layout_passes=False,
        ),
    )
    def gather_k(ppi_hbm, lut_hbm, out_hbm,
                 idx0_v, idx1_v, lut0_v, lut1_v, out0_v, out1_v):
        wid = lax.axis_index("s") * nc + lax.axis_index("c")
        base = wid * chunk_e
        pltpu.sync_copy(lut_hbm.at[0, :], lut0_v)
        pltpu.sync_copy(lut_hbm.at[1, :], lut1_v)
        pltpu.sync_copy(ppi_hbm.at[0, pl.ds(base, chunk_e)], idx0_v)
        pltpu.sync_copy(ppi_hbm.at[1, pl.ds(base, chunk_e)], idx1_v)

        def body(i, _):
            sl = pl.ds(i * _LANES, _LANES)
            out0_v[sl] = plsc.load_gather(lut0_v, [idx0_v[sl]])
            out1_v[sl] = plsc.load_gather(lut1_v, [idx1_v[sl]])
            return 0

        lax.fori_loop(0, chunk_e // _LANES, body, 0)
        pltpu.sync_copy(out0_v, out_hbm.at[0, pl.ds(base, chunk_e)])
        pltpu.sync_copy(out1_v, out_hbm.at[1, pl.ds(base, chunk_e)])

    return gather_k(ppi_t, lut)


def kernel(feature, ppi, kernel, bias):
    n_edges = ppi.shape[0]
    lut = _build_lut(feature, kernel, bias)
    out_t = _gather_lut(ppi.T, lut, n_edges)
    return out_t.T[None]


# speedup vs baseline: 9.7879x; 1.1283x over previous
import functools

import jax
import jax.numpy as jnp
from jax import lax
from jax.experimental import pallas as pl
from jax.experimental.pallas import tpu as pltpu
from jax.experimental.pallas import tpu_sc as plsc

_N_ROWS = 10000
_N_UNITS = 2
_ROW_BLK = 1000
_LANES = 16


def _lut_body(f_ref, k_ref, b_ref, o_ref):
    rs = jnp.sum(jnp.abs(f_ref[0]), axis=1)
    o_ref[...] = jax.nn.sigmoid(rs[None, :] * k_ref[...] + b_ref[...])


def _build_lut(feature, kern, bias):
    return pl.pallas_call(
        _lut_body,
        out_shape=jax.ShapeDtypeStruct((_N_UNITS, _N_ROWS), jnp.float32),
    )(feature, kern.reshape(_N_UNITS, 1), bias.reshape(_N_UNITS, 1))


def _gather_lut(ppi_t, lut, n_edges):
    info = plsc.get_sparse_core_info()
    nc, ns = info.num_cores, info.num_subcores
    nw = nc * ns
    chunk_e = n_edges // nw

    mesh = plsc.VectorSubcoreMesh(core_axis_name="c", subcore_axis_name="s")

    @functools.partial(
        pl.kernel,
        mesh=mesh,
        out_type=jax.ShapeDtypeStruct((_N_UNITS, n_edges), jnp.float32),
        scratch_types=[
            pltpu.VMEM((chunk_e,), jnp.int32),
            pltpu.VMEM((chunk_e,), jnp.int32),
            pltpu.VMEM((_N_ROWS,), jnp.float32),
            pltpu.VMEM((_N_ROWS,), jnp.float32),
            pltpu.VMEM((chunk_e,), jnp.float32),
            pltpu.VMEM((chunk_e,), jnp.float32),
            pltpu.SemaphoreType.DMA,
            pltpu.SemaphoreType.DMA,
            pltpu.SemaphoreType.DMA,
            pltpu.SemaphoreType.DMA,
        ],
        compiler_params=pltpu.CompilerParams(
            use_tc_tiling_on_sc=False,
            needs_layout_passes=False,
        ),
    )
    def gather_k(ppi_hbm, lut_hbm, out_hbm,
                 idx0_v, idx1_v, lut0_v, lut1_v, out0_v, out1_v,
                 sem0, sem1, sem2, sem3):
        wid = lax.axis_index("s") * nc + lax.axis_index("c")
        base = wid * chunk_e
        c0 = pltpu.async_copy(lut_hbm.at[0, :], lut0_v, sem0)
        c1 = pltpu.async_copy(lut_hbm.at[1, :], lut1_v, sem1)
        c2 = pltpu.async_copy(ppi_hbm.at[0, pl.ds(base, chunk_e)], idx0_v, sem2)
        c3 = pltpu.async_copy(ppi_hbm.at[1, pl.ds(base, chunk_e)], idx1_v, sem3)
        c0.wait()
        c1.wait()
        c2.wait()
        c3.wait()

        @plsc.parallel_loop(0, chunk_e // _LANES, unroll=8)
        def body(i):
            sl = pl.ds(i * _LANES, _LANES)
            out0_v[sl] = plsc.load_gather(lut0_v, [idx0_v[sl]])
            out1_v[sl] = plsc.load_gather(lut1_v, [idx1_v[sl]])

        c4 = pltpu.async_copy(out0_v, out_hbm.at[0, pl.ds(base, chunk_e)], sem0)
        c5 = pltpu.async_copy(out1_v, out_hbm.at[1, pl.ds(base, chunk_e)], sem1)
        c4.wait()
        c5.wait()

    return gather_k(ppi_t, lut)


def kernel(feature, ppi, kernel, bias):
    n_edges = ppi.shape[0]
    lut = _build_lut(feature, kernel, bias)
    out_t = _gather_lut(ppi.T, lut, n_edges)
    return out_t.T[None]


# speedup vs baseline: 10.6841x; 1.0916x over previous
import functools

import jax
import jax.numpy as jnp
from jax import lax
from jax.experimental import pallas as pl
from jax.experimental.pallas import tpu as pltpu
from jax.experimental.pallas import tpu_sc as plsc

_N_ROWS = 10000
_N_UNITS = 2
_ROW_BLK = 1000
_LANES = 16


def _lut_body(f_ref, k_ref, b_ref, o_ref):
    absf = jnp.abs(f_ref[0])
    ones = jnp.ones((8, 128), jnp.float32)
    rs8 = lax.dot_general(ones, absf, (((1,), (1,)), ((), ())),
                          precision=lax.Precision.HIGHEST)
    o_ref[...] = jax.nn.sigmoid(rs8[:_N_UNITS] * k_ref[...] + b_ref[...])


def _build_lut(feature, kern, bias):
    return pl.pallas_call(
        _lut_body,
        out_shape=jax.ShapeDtypeStruct((_N_UNITS, _N_ROWS), jnp.float32),
    )(feature, kern.reshape(_N_UNITS, 1), bias.reshape(_N_UNITS, 1))


def _gather_lut(ppi_t, lut, n_edges):
    info = plsc.get_sparse_core_info()
    nc, ns = info.num_cores, info.num_subcores
    nw = nc * ns
    chunk_e = n_edges // nw

    mesh = plsc.VectorSubcoreMesh(core_axis_name="c", subcore_axis_name="s")

    @functools.partial(
        pl.kernel,
        mesh=mesh,
        out_type=jax.ShapeDtypeStruct((_N_UNITS, n_edges), jnp.float32),
        scratch_types=[
            pltpu.VMEM((chunk_e,), jnp.int32),
            pltpu.VMEM((chunk_e,), jnp.int32),
            pltpu.VMEM((_N_ROWS,), jnp.float32),
            pltpu.VMEM((_N_ROWS,), jnp.float32),
            pltpu.VMEM((chunk_e,), jnp.float32),
            pltpu.VMEM((chunk_e,), jnp.float32),
            pltpu.SemaphoreType.DMA,
            pltpu.SemaphoreType.DMA,
            pltpu.SemaphoreType.DMA,
            pltpu.SemaphoreType.DMA,
        ],
        compiler_params=pltpu.CompilerParams(
            use_tc_tiling_on_sc=False,
            needs_layout_passes=False,
        ),
    )
    def gather_k(ppi_hbm, lut_hbm, out_hbm,
                 idx0_v, idx1_v, lut0_v, lut1_v, out0_v, out1_v,
                 sem0, sem1, sem2, sem3):
        wid = lax.axis_index("s") * nc + lax.axis_index("c")
        base = wid * chunk_e
        c0 = pltpu.async_copy(lut_hbm.at[0, :], lut0_v, sem0)
        c1 = pltpu.async_copy(lut_hbm.at[1, :], lut1_v, sem1)
        c2 = pltpu.async_copy(ppi_hbm.at[0, pl.ds(base, chunk_e)], idx0_v, sem2)
        c3 = pltpu.async_copy(ppi_hbm.at[1, pl.ds(base, chunk_e)], idx1_v, sem3)
        c0.wait()
        c1.wait()
        c2.wait()
        c3.wait()

        @plsc.parallel_loop(0, chunk_e // _LANES, unroll=8)
        def body(i):
            sl = pl.ds(i * _LANES, _LANES)
            out0_v[sl] = plsc.load_gather(lut0_v, [idx0_v[sl]])
            out1_v[sl] = plsc.load_gather(lut1_v, [idx1_v[sl]])

        c4 = pltpu.async_copy(out0_v, out_hbm.at[0, pl.ds(base, chunk_e)], sem0)
        c5 = pltpu.async_copy(out1_v, out_hbm.at[1, pl.ds(base, chunk_e)], sem1)
        c4.wait()
        c5.wait()

    return gather_k(ppi_t, lut)


def kernel(feature, ppi, kernel, bias):
    n_edges = ppi.shape[0]
    lut = _build_lut(feature, kernel, bias)
    out_t = _gather_lut(ppi.T, lut, n_edges)
    return out_t.T[None]


# speedup vs baseline: 10.7905x; 1.0100x over previous
import functools

import jax
import jax.numpy as jnp
from jax import lax
from jax.experimental import pallas as pl
from jax.experimental.pallas import tpu as pltpu
from jax.experimental.pallas import tpu_sc as plsc

_N_ROWS = 10000
_N_UNITS = 2
_ROW_BLK = 1000
_LANES = 16


def _lut_body(f_ref, k_ref, b_ref, o_ref):
    absf = jnp.abs(f_ref[0])
    ones = jnp.ones((8, 128), jnp.float32)
    rs8 = lax.dot_general(ones, absf, (((1,), (1,)), ((), ())),
                          precision=lax.Precision.HIGHEST)
    o_ref[...] = jax.nn.sigmoid(rs8[:_N_UNITS] * k_ref[...] + b_ref[...])


def _build_lut(feature, kern, bias):
    return pl.pallas_call(
        _lut_body,
        out_shape=jax.ShapeDtypeStruct((_N_UNITS, _N_ROWS), jnp.float32),
    )(feature, kern.reshape(_N_UNITS, 1), bias.reshape(_N_UNITS, 1))


def _gather_lut(ppi_t, lut, n_edges):
    info = plsc.get_sparse_core_info()
    nc, ns = info.num_cores, info.num_subcores
    nw = nc * ns
    chunk_e = n_edges // nw

    mesh = plsc.VectorSubcoreMesh(core_axis_name="c", subcore_axis_name="s")

    @functools.partial(
        pl.kernel,
        mesh=mesh,
        out_type=jax.ShapeDtypeStruct((_N_UNITS, n_edges), jnp.float32),
        scratch_types=[
            pltpu.VMEM((chunk_e,), jnp.int32),
            pltpu.VMEM((chunk_e,), jnp.int32),
            pltpu.VMEM((_N_ROWS,), jnp.float32),
            pltpu.VMEM((_N_ROWS,), jnp.float32),
            pltpu.VMEM((chunk_e,), jnp.float32),
            pltpu.VMEM((chunk_e,), jnp.float32),
            pltpu.SemaphoreType.DMA,
            pltpu.SemaphoreType.DMA,
            pltpu.SemaphoreType.DMA,
            pltpu.SemaphoreType.DMA,
        ],
        compiler_params=pltpu.CompilerParams(
            use_tc_tiling_on_sc=False,
            needs_layout_passes=False,
        ),
    )
    def gather_k(ppi_hbm, lut_hbm, out_hbm,
                 idx0_v, idx1_v, lut0_v, lut1_v, out0_v, out1_v,
                 sem0, sem1, sem2, sem3):
        wid = lax.axis_index("s") * nc + lax.axis_index("c")
        base = wid * chunk_e
        c0 = pltpu.async_copy(lut_hbm.at[0, :], lut0_v, sem0)
        c2 = pltpu.async_copy(ppi_hbm.at[0, pl.ds(base, chunk_e)], idx0_v, sem2)
        c1 = pltpu.async_copy(lut_hbm.at[1, :], lut1_v, sem1)
        c3 = pltpu.async_copy(ppi_hbm.at[1, pl.ds(base, chunk_e)], idx1_v, sem3)
        c0.wait()
        c2.wait()

        @plsc.parallel_loop(0, chunk_e // _LANES, unroll=8)
        def body0(i):
            sl = pl.ds(i * _LANES, _LANES)
            out0_v[sl] = plsc.load_gather(lut0_v, [idx0_v[sl]])

        c4 = pltpu.async_copy(out0_v, out_hbm.at[0, pl.ds(base, chunk_e)], sem0)
        c1.wait()
        c3.wait()

        @plsc.parallel_loop(0, chunk_e // _LANES, unroll=8)
        def body1(i):
            sl = pl.ds(i * _LANES, _LANES)
            out1_v[sl] = plsc.load_gather(lut1_v, [idx1_v[sl]])

        c5 = pltpu.async_copy(out1_v, out_hbm.at[1, pl.ds(base, chunk_e)], sem1)
        c4.wait()
        c5.wait()

    return gather_k(ppi_t, lut)


def kernel(feature, ppi, kernel, bias):
    n_edges = ppi.shape[0]
    lut = _build_lut(feature, kernel, bias)
    out_t = _gather_lut(ppi.T, lut, n_edges)
    return out_t.T[None]
